# baseline (speedup 1.0000x reference)
import functools

import jax
import jax.numpy as jnp
from jax import lax
from jax.experimental import pallas as pl
from jax.experimental.pallas import tpu as pltpu
from jax.experimental.pallas import tpu_sc as plsc

B = 8
N = 64
T = 16
TD = 16
D = T * TD
R = 64
PAD_TOKEN = 3
BN = B * N


def _sc_masks_body(pre_hbm, keep_hbm, rm_hbm, oh0_hbm, oh1_hbm, ohT_hbm,
                   pre_v, keep_v, rm_v, oh0_v, oh1_v, ohT_v):
    b = lax.axis_index("s") * 2 + lax.axis_index("c")

    @pl.when(b < B)
    def _():
        pltpu.sync_copy(pre_hbm.at[b], pre_v)
        auxvec = pre_v[pl.ds(2 * N, 16)]
        lane = lax.iota(jnp.int32, 16)
        c0s = jnp.sum(jnp.where(lane == 0, auxvec, 0))
        c1s = jnp.sum(jnp.where(lane == 1, auxvec, 0))
        tgs = jnp.sum(jnp.where(lane == 2, auxvec, 0))
        nodup = c0s != c1s
        carry = jnp.int32(0)
        for c in range(N // 16):
            sl = pl.ds(c * 16, 16)
            ids = pre_v[sl]
            pad = (ids == PAD_TOKEN).astype(jnp.int32)
            incl = plsc.cumsum(pad)
            prev = incl - pad + carry
            keep_v[sl] = (prev == 0).astype(jnp.float32)
            carry = carry + jnp.sum(pad)
            node = lane + (c * 16)
            oh0_v[sl] = (node == c0s).astype(jnp.float32)
            oh1_v[sl] = ((node == c1s) & nodup).astype(jnp.float32)
            ohT_v[sl] = (node == tgs).astype(jnp.float32)
            rm_v[sl] = jnp.zeros((16,), jnp.float32)
        for c in range(N // 16):
            idx = pre_v[pl.ds(N + c * 16, 16)]
            plsc.store_scatter(rm_v, [idx], jnp.ones((16,), jnp.float32))
        pltpu.sync_copy(keep_v, keep_hbm.at[pl.ds(N * b, N)])
        pltpu.sync_copy(rm_v, rm_hbm.at[pl.ds(N * b, N)])
        pltpu.sync_copy(oh0_v, oh0_hbm.at[pl.ds(N * b, N)])
        pltpu.sync_copy(oh1_v, oh1_hbm.at[pl.ds(N * b, N)])
        pltpu.sync_copy(ohT_v, ohT_hbm.at[pl.ds(N * b, N)])


def _nt_dot(x, w, precision):
    return lax.dot_general(x, w, (((1,), (1,)), ((), ())), precision=precision)


def _bsum(colmask, x):
    return jnp.sum((colmask * x).reshape(B, N, x.shape[-1]), axis=1)


def _brow(x8, c):
    return jnp.broadcast_to(x8[:, None, :], (B, N, c)).reshape(BN, c)


def _tc_gru_body(E_ref, samp_ref, keepc_ref, rmc_ref, oh0c_ref, oh1c_ref,
                 ohTc_ref, wihb_ref, whhb_ref, wihf_ref, whhf_ref, out_ref):
    hi = lax.Precision.HIGHEST
    med = lax.Precision.DEFAULT
    E = E_ref[:]
    samp = samp_ref[:]
    keepc = keepc_ref[:]
    rmc = rmc_ref[:]
    oh0c = oh0c_ref[:]
    oh1c = oh1c_ref[:]
    ohTc = ohTc_ref[:]
    V = keepc * (E + ohTc * (_brow(samp, D) - E))

    def gates(gi, gh, h):
        r = jax.nn.sigmoid(gi[:, 0:D] + gh[:, 0:D])
        z = jax.nn.sigmoid(gi[:, D:2 * D] + gh[:, D:2 * D])
        n = jnp.tanh(gi[:, 2 * D:3 * D] + r * gh[:, 2 * D:3 * D])
        return (1.0 - z) * n + z * h

    GI = _nt_dot(V, wihb_ref[:], med)
    h0 = _bsum(oh0c, V)
    h1 = _bsum(oh1c, V)
    gh0 = _brow(_nt_dot(h0, whhb_ref[:], hi), 3 * D)
    gh1 = _brow(_nt_dot(h1, whhb_ref[:], hi), 3 * D)
    out0 = gates(GI, gh0, _brow(h0, D))
    out1 = gates(GI, gh1, _brow(h1, D))
    u0 = _bsum(rmc, out0)
    u1 = _bsum(rmc, out1)
    Vp = V + keepc * (oh0c * _brow(u0, D) + oh1c * _brow(u1, D))
    GH = _nt_dot(Vp, whhf_ref[:], med)
    x0 = _bsum(oh0c, Vp)
    x1 = _bsum(oh1c, Vp)
    gx0 = _brow(_nt_dot(x0, wihf_ref[:], hi), 3 * D)
    gx1 = _brow(_nt_dot(x1, wihf_ref[:], hi), 3 * D)
    w0 = _bsum(rmc, gates(gx0, GH, Vp))
    w1 = _bsum(rmc, gates(gx1, GH, Vp))
    sel0 = _bsum(ohTc, oh0c)
    sel1 = _bsum(ohTc, oh1c)
    out_ref[:] = samp + sel0 * (u0 + w0) + sel1 * (u1 + w1)


def kernel(tgt_idx, relation_at, input_split_ids, input_split_encode, sample,
           W_ih_fwd, W_hh_fwd, b_ih_fwd, b_hh_fwd,
           W_ih_bwd, W_hh_bwd, b_ih_bwd, b_hh_bwd, K):
    i32 = jnp.int32
    f32 = jnp.float32
    pre = jnp.concatenate(
        [input_split_ids[:, :, 0].astype(i32),
         relation_at[:, :, 1].astype(i32),
         relation_at[:, 0, :].astype(i32),
         tgt_idx[:, None].astype(i32),
         jnp.zeros((B, 13), i32)], axis=1)

    mesh = plsc.VectorSubcoreMesh(core_axis_name="c", subcore_axis_name="s",
                                  num_cores=2, num_subcores=16)
    mask_t = jax.ShapeDtypeStruct((BN,), f32)
    keepf, rmf, oh0f, oh1f, ohTf = pl.kernel(
        _sc_masks_body,
        out_type=(mask_t,) * 5,
        mesh=mesh,
        compiler_params=pltpu.CompilerParams(needs_layout_passes=False),
        scratch_types=[
            pltpu.VMEM((144,), i32),
            pltpu.VMEM((N,), f32), pltpu.VMEM((N,), f32), pltpu.VMEM((N,), f32),
            pltpu.VMEM((N,), f32), pltpu.VMEM((N,), f32),
        ],
    )(pre)

    keepc = keepf.reshape(BN, 1)
    rmc = rmf.reshape(BN, 1)
    oh0c = oh0f.reshape(BN, 1)
    oh1c = oh1f.reshape(BN, 1)
    ohTc = ohTf.reshape(BN, 1)

    E = input_split_encode.reshape(BN, D)
    samp8 = sample.reshape(B, D)
    out8 = pl.pallas_call(
        _tc_gru_body,
        out_shape=jax.ShapeDtypeStruct((B, D), f32),
        in_specs=[pl.BlockSpec(memory_space=pltpu.VMEM)] * 11,
        out_specs=pl.BlockSpec(memory_space=pltpu.VMEM),
    )(E, samp8, keepc, rmc, oh0c, oh1c, ohTc,
      W_ih_bwd, W_hh_bwd, W_ih_fwd, W_hh_fwd)
    return out8.reshape(B, T, TD)


# speedup vs baseline: 1.2189x; 1.2189x over previous
import functools

import jax
import jax.numpy as jnp
from jax import lax
from jax.experimental import pallas as pl
from jax.experimental.pallas import tpu as pltpu
from jax.experimental.pallas import tpu_sc as plsc

B = 8
N = 64
T = 16
TD = 16
D = T * TD
R = 64
PAD_TOKEN = 3
BN = B * N


def _sc_masks_body(pre_hbm, out_hbm,
                   pre_v, keep_v, rm_v, oh0_v, oh1_v, ohT_v):
    b = lax.axis_index("s") * 2 + lax.axis_index("c")

    @pl.when(b < B)
    def _():
        pltpu.sync_copy(pre_hbm.at[b], pre_v)
        auxvec = pre_v[pl.ds(2 * N, 16)]
        lane = lax.iota(jnp.int32, 16)
        c0s = jnp.sum(jnp.where(lane == 0, auxvec, 0))
        c1s = jnp.sum(jnp.where(lane == 1, auxvec, 0))
        tgs = jnp.sum(jnp.where(lane == 2, auxvec, 0))
        nodup = c0s != c1s
        carry = jnp.int32(0)
        for c in range(N // 16):
            sl = pl.ds(c * 16, 16)
            ids = pre_v[sl]
            pad = (ids == PAD_TOKEN).astype(jnp.int32)
            incl = plsc.cumsum(pad)
            prev = incl - pad + carry
            keep_v[sl] = (prev == 0).astype(jnp.float32)
            carry = carry + jnp.sum(pad)
            node = lane + (c * 16)
            oh0_v[sl] = (node == c0s).astype(jnp.float32)
            oh1_v[sl] = ((node == c1s) & nodup).astype(jnp.float32)
            ohT_v[sl] = (node == tgs).astype(jnp.float32)
            rm_v[sl] = jnp.zeros((16,), jnp.float32)
        for c in range(N // 16):
            idx = pre_v[pl.ds(N + c * 16, 16)]
            plsc.store_scatter(rm_v, [idx], jnp.ones((16,), jnp.float32))
        pltpu.sync_copy(keep_v, out_hbm.at[0 * B + b])
        pltpu.sync_copy(rm_v, out_hbm.at[1 * B + b])
        pltpu.sync_copy(oh0_v, out_hbm.at[2 * B + b])
        pltpu.sync_copy(oh1_v, out_hbm.at[3 * B + b])
        pltpu.sync_copy(ohT_v, out_hbm.at[4 * B + b])


def _nt_dot(x, w, precision):
    return lax.dot_general(x, w, (((1,), (1,)), ((), ())), precision=precision)


def _bsum(colmask, x):
    return jnp.sum((colmask * x).reshape(B, N, x.shape[-1]), axis=1)


def _brow(x8, c):
    return jnp.broadcast_to(x8[:, None, :], (B, N, c)).reshape(BN, c)


def _tc_gru_body(E_ref, samp_ref, masks_ref, wihb_ref, whhb_ref,
                 wihf_ref, whhf_ref, out_ref):
    hi = lax.Precision.HIGHEST
    med = lax.Precision.DEFAULT
    E = E_ref[:]
    samp = samp_ref[:]
    masks = masks_ref[:]
    Pc = (lax.broadcasted_iota(jnp.int32, (BN, B), 0) // N
          == lax.broadcasted_iota(jnp.int32, (BN, B), 1)).astype(jnp.float32)
    Xsel = (lax.broadcasted_iota(jnp.int32, (BN, N), 0) % N
            == lax.broadcasted_iota(jnp.int32, (BN, N), 1)).astype(jnp.float32)

    def col(m8):
        return jnp.sum(jnp.dot(Pc, m8) * Xsel, axis=1, keepdims=True)

    keepc = col(masks[0:B])
    rmc = col(masks[B:2 * B])
    oh0c = col(masks[2 * B:3 * B])
    oh1c = col(masks[3 * B:4 * B])
    ohTc = col(masks[4 * B:5 * B])
    V = keepc * (E + ohTc * (_brow(samp, D) - E))

    def gates(gi, gh, h):
        r = jax.nn.sigmoid(gi[:, 0:D] + gh[:, 0:D])
        z = jax.nn.sigmoid(gi[:, D:2 * D] + gh[:, D:2 * D])
        n = jnp.tanh(gi[:, 2 * D:3 * D] + r * gh[:, 2 * D:3 * D])
        return (1.0 - z) * n + z * h

    GI = _nt_dot(V, wihb_ref[:], med)
    h0 = _bsum(oh0c, V)
    h1 = _bsum(oh1c, V)
    gh0 = _brow(_nt_dot(h0, whhb_ref[:], hi), 3 * D)
    gh1 = _brow(_nt_dot(h1, whhb_ref[:], hi), 3 * D)
    out0 = gates(GI, gh0, _brow(h0, D))
    out1 = gates(GI, gh1, _brow(h1, D))
    u0 = _bsum(rmc, out0)
    u1 = _bsum(rmc, out1)
    Vp = V + keepc * (oh0c * _brow(u0, D) + oh1c * _brow(u1, D))
    GH = _nt_dot(Vp, whhf_ref[:], med)
    x0 = _bsum(oh0c, Vp)
    x1 = _bsum(oh1c, Vp)
    gx0 = _brow(_nt_dot(x0, wihf_ref[:], hi), 3 * D)
    gx1 = _brow(_nt_dot(x1, wihf_ref[:], hi), 3 * D)
    w0 = _bsum(rmc, gates(gx0, GH, Vp))
    w1 = _bsum(rmc, gates(gx1, GH, Vp))
    sel0 = _bsum(ohTc, oh0c)
    sel1 = _bsum(ohTc, oh1c)
    out_ref[:] = samp + sel0 * (u0 + w0) + sel1 * (u1 + w1)


def kernel(tgt_idx, relation_at, input_split_ids, input_split_encode, sample,
           W_ih_fwd, W_hh_fwd, b_ih_fwd, b_hh_fwd,
           W_ih_bwd, W_hh_bwd, b_ih_bwd, b_hh_bwd, K):
    i32 = jnp.int32
    f32 = jnp.float32
    pre = jnp.concatenate(
        [input_split_ids[:, :, 0].astype(i32),
         relation_at[:, :, 1].astype(i32),
         relation_at[:, 0, :].astype(i32),
         tgt_idx[:, None].astype(i32),
         jnp.zeros((B, 13), i32)], axis=1)

    mesh = plsc.VectorSubcoreMesh(core_axis_name="c", subcore_axis_name="s",
                                  num_cores=2, num_subcores=16)
    masks = pl.kernel(
        _sc_masks_body,
        out_type=jax.ShapeDtypeStruct((5 * B, N), f32),
        mesh=mesh,
        compiler_params=pltpu.CompilerParams(needs_layout_passes=False),
        scratch_types=[
            pltpu.VMEM((144,), i32),
            pltpu.VMEM((N,), f32), pltpu.VMEM((N,), f32), pltpu.VMEM((N,), f32),
            pltpu.VMEM((N,), f32), pltpu.VMEM((N,), f32),
        ],
    )(pre)

    E = input_split_encode.reshape(BN, D)
    samp8 = sample.reshape(B, D)
    out8 = pl.pallas_call(
        _tc_gru_body,
        out_shape=jax.ShapeDtypeStruct((B, D), f32),
        in_specs=[pl.BlockSpec(memory_space=pltpu.VMEM)] * 7,
        out_specs=pl.BlockSpec(memory_space=pltpu.VMEM),
    )(E, samp8, masks, W_ih_bwd, W_hh_bwd, W_ih_fwd, W_hh_fwd)
    return out8.reshape(B, T, TD)
